# baseline (speedup 1.0000x reference)
import functools

import jax
import jax.numpy as jnp
from jax import lax
from jax.experimental import pallas as pl
from jax.experimental.pallas import tpu as pltpu
from jax.experimental.pallas import tpu_sc as plsc


def _sc_gather(emb, y):
    B = y.shape[0]
    _, D = emb.shape
    info = plsc.get_sparse_core_info()
    NC, NS = info.num_cores, info.num_subcores
    NW = NC * NS
    b_per_w = B // NW
    mesh = plsc.VectorSubcoreMesh(core_axis_name="c", subcore_axis_name="s")

    @functools.partial(
        pl.kernel,
        mesh=mesh,
        out_type=jax.ShapeDtypeStruct((B, D), jnp.float32),
        scratch_types=[
            pltpu.VMEM((b_per_w,), jnp.int32),
            pltpu.VMEM((b_per_w, D), jnp.float32),
            pltpu.SemaphoreType.DMA,
        ],
    )
    def gather_kernel(emb_hbm, y_hbm, out_hbm, idx_v, rows_v, sem):
        wid = lax.axis_index("s") * NC + lax.axis_index("c")
        base = wid * b_per_w
        pltpu.sync_copy(y_hbm.at[pl.ds(base, b_per_w)], idx_v)
        pltpu.async_copy(emb_hbm.at[idx_v], rows_v, sem).wait()
        pltpu.sync_copy(rows_v, out_hbm.at[pl.ds(base, b_per_w)])

    return gather_kernel(emb, y)


def _fused_body(y_ref, x_ref, *rest):
    o_ref = rest[-1]
    e_refs = rest[:-1]
    rows = jnp.concatenate([er[...] for er in e_refs], axis=0)
    o_ref[...] = x_ref[...] + rows


def _tc_fused_add(x, y, emb, bb=8):
    B, S, D = x.shape
    emb3 = emb.reshape(emb.shape[0], 1, D)

    def emap(j):
        return lambda i, yr: (yr[i * bb + j], 0, 0)

    return pl.pallas_call(
        _fused_body,
        grid_spec=pltpu.PrefetchScalarGridSpec(
            num_scalar_prefetch=1,
            grid=(B // bb,),
            in_specs=[pl.BlockSpec((bb, S, D), lambda i, yr: (i, 0, 0))]
            + [pl.BlockSpec((1, 1, D), emap(j)) for j in range(bb)],
            out_specs=pl.BlockSpec((bb, S, D), lambda i, yr: (i, 0, 0)),
        ),
        out_shape=jax.ShapeDtypeStruct((B, S, D), x.dtype),
    )(y, x, *([emb3] * bb))


def _add_body(x_ref, e_ref, o_ref):
    o_ref[...] = x_ref[...] + e_ref[...][:, None, :]


def _tc_add(x, e):
    B, S, D = x.shape
    BB = 128
    return pl.pallas_call(
        _add_body,
        grid=(B // BB,),
        in_specs=[
            pl.BlockSpec((BB, S, D), lambda i: (i, 0, 0)),
            pl.BlockSpec((BB, D), lambda i: (i, 0)),
        ],
        out_specs=pl.BlockSpec((BB, S, D), lambda i: (i, 0, 0)),
        out_shape=jax.ShapeDtypeStruct((B, S, D), x.dtype),
    )(x, e)


def _sc_full(x2, y, emb):
    B, SD = x2.shape
    _, D = emb.shape
    info = plsc.get_sparse_core_info()
    NC, NS, L = info.num_cores, info.num_subcores, info.num_lanes
    NW = NC * NS
    bw = B // NW
    NBUF = 3
    mesh = plsc.VectorSubcoreMesh(core_axis_name="c", subcore_axis_name="s")

    @functools.partial(
        pl.kernel,
        mesh=mesh,
        out_type=jax.ShapeDtypeStruct((B, SD), jnp.float32),
        scratch_types=[
            pltpu.VMEM((bw,), jnp.int32),
            pltpu.VMEM((bw, D), jnp.float32),
            pltpu.SemaphoreType.DMA,
        ]
        + [pltpu.VMEM((SD,), jnp.float32) for _ in range(NBUF)]
        + [pltpu.SemaphoreType.DMA for _ in range(NBUF)]
        + [pltpu.SemaphoreType.DMA for _ in range(NBUF)],
    )
    def body(x_hbm, y_hbm, emb_hbm, out_hbm, idx_v, rows_v, gsem, *bufsems):
        bufs = bufsems[:NBUF]
        isems = bufsems[NBUF:2 * NBUF]
        osems = bufsems[2 * NBUF:]
        wid = lax.axis_index("s") * NC + lax.axis_index("c")
        base = wid * bw
        pltpu.sync_copy(y_hbm.at[pl.ds(base, bw)], idx_v)
        pltpu.async_copy(emb_hbm.at[idx_v], rows_v, gsem).wait()

        in_cp = [None] * NBUF
        out_cp = [None] * NBUF
        for b in range(bw + 1):
            r = b % NBUF
            if b < bw:
                if out_cp[r] is not None:
                    out_cp[r].wait()
                    out_cp[r] = None
                in_cp[r] = pltpu.async_copy(
                    x_hbm.at[base + b], bufs[r], isems[r])
            if b >= 1:
                pb = b - 1
                pr = pb % NBUF
                in_cp[pr].wait()
                ev = [rows_v[pb, pl.ds(j * L, L)] for j in range(D // L)]
                buf = bufs[pr]

                def add_row(i, _, buf=buf, ev=ev):
                    off = i * D
                    for j in range(D // L):
                        sl = pl.ds(off + j * L, L)
                        buf[sl] = buf[sl] + ev[j]
                    return 0

                lax.fori_loop(0, SD // D, add_row, 0)
                out_cp[pr] = pltpu.async_copy(
                    bufs[pr], out_hbm.at[base + pb], osems[pr])
        for r in range(NBUF):
            if out_cp[r] is not None:
                out_cp[r].wait()

    return body(x2, y, emb)


def kernel(x, y, emb):
    y = y.astype(jnp.int32)
    B, S, D = x.shape
    out2 = _sc_full(x.reshape(B, S * D), y, emb)
    return out2.reshape(B, S, D)


# speedup vs baseline: 2.5405x; 2.5405x over previous
import functools

import jax
import jax.numpy as jnp
from jax import lax
from jax.experimental import pallas as pl
from jax.experimental.pallas import tpu as pltpu
from jax.experimental.pallas import tpu_sc as plsc


def _sc_gather(emb, y):
    B = y.shape[0]
    _, D = emb.shape
    info = plsc.get_sparse_core_info()
    NC, NS = info.num_cores, info.num_subcores
    NW = NC * NS
    b_per_w = B // NW
    mesh = plsc.VectorSubcoreMesh(core_axis_name="c", subcore_axis_name="s")

    @functools.partial(
        pl.kernel,
        mesh=mesh,
        out_type=jax.ShapeDtypeStruct((B, D), jnp.float32),
        scratch_types=[
            pltpu.VMEM((b_per_w,), jnp.int32),
            pltpu.VMEM((b_per_w, D), jnp.float32),
            pltpu.SemaphoreType.DMA,
        ],
    )
    def gather_kernel(emb_hbm, y_hbm, out_hbm, idx_v, rows_v, sem):
        wid = lax.axis_index("s") * NC + lax.axis_index("c")
        base = wid * b_per_w
        pltpu.sync_copy(y_hbm.at[pl.ds(base, b_per_w)], idx_v)
        pltpu.async_copy(emb_hbm.at[idx_v], rows_v, sem).wait()
        pltpu.sync_copy(rows_v, out_hbm.at[pl.ds(base, b_per_w)])

    return gather_kernel(emb, y)


def _fused_body(y_ref, x_ref, *rest):
    o_ref = rest[-1]
    e_refs = rest[:-1]
    rows = jnp.concatenate([er[...] for er in e_refs], axis=0)
    o_ref[...] = x_ref[...] + rows


def _tc_fused_add(x, y, emb, bb=8):
    B, S, D = x.shape
    emb3 = emb.reshape(emb.shape[0], 1, D)

    def emap(j):
        return lambda i, yr: (yr[i * bb + j], 0, 0)

    return pl.pallas_call(
        _fused_body,
        grid_spec=pltpu.PrefetchScalarGridSpec(
            num_scalar_prefetch=1,
            grid=(B // bb,),
            in_specs=[pl.BlockSpec((bb, S, D), lambda i, yr: (i, 0, 0))]
            + [pl.BlockSpec((1, 1, D), emap(j)) for j in range(bb)],
            out_specs=pl.BlockSpec((bb, S, D), lambda i, yr: (i, 0, 0)),
        ),
        out_shape=jax.ShapeDtypeStruct((B, S, D), x.dtype),
    )(y, x, *([emb3] * bb))


def _add_body(x_ref, e_ref, o_ref):
    o_ref[...] = x_ref[...] + e_ref[...][:, None, :]


def _tc_add(x, e):
    B, S, D = x.shape
    BB = 128
    return pl.pallas_call(
        _add_body,
        grid=(B // BB,),
        in_specs=[
            pl.BlockSpec((BB, S, D), lambda i: (i, 0, 0)),
            pl.BlockSpec((BB, D), lambda i: (i, 0)),
        ],
        out_specs=pl.BlockSpec((BB, S, D), lambda i: (i, 0, 0)),
        out_shape=jax.ShapeDtypeStruct((B, S, D), x.dtype),
    )(x, e)


def _sc_full(x, y, emb):
    B, S, D = x.shape
    info = plsc.get_sparse_core_info()
    NC, NS, L = info.num_cores, info.num_subcores, info.num_lanes
    NW = NC * NS
    bw = B // NW
    NBUF = 3
    mesh = plsc.VectorSubcoreMesh(core_axis_name="c", subcore_axis_name="s")

    @functools.partial(
        pl.kernel,
        mesh=mesh,
        out_type=jax.ShapeDtypeStruct((B, S, D), jnp.float32),
        scratch_types=[
            pltpu.VMEM((bw,), jnp.int32),
            pltpu.VMEM((bw, D), jnp.float32),
            pltpu.SemaphoreType.DMA,
        ]
        + [pltpu.VMEM((S, D), jnp.float32) for _ in range(NBUF)]
        + [pltpu.SemaphoreType.DMA for _ in range(NBUF)]
        + [pltpu.SemaphoreType.DMA for _ in range(NBUF)],
    )
    def body(x_hbm, y_hbm, emb_hbm, out_hbm, idx_v, rows_v, gsem, *bufsems):
        bufs = bufsems[:NBUF]
        isems = bufsems[NBUF:2 * NBUF]
        osems = bufsems[2 * NBUF:]
        wid = lax.axis_index("s") * NC + lax.axis_index("c")
        base = wid * bw
        pltpu.sync_copy(y_hbm.at[pl.ds(base, bw)], idx_v)
        pltpu.async_copy(emb_hbm.at[idx_v], rows_v, gsem).wait()

        in_cp = [None] * NBUF
        out_cp = [None] * NBUF
        for b in range(bw + 1):
            r = b % NBUF
            if b < bw:
                if out_cp[r] is not None:
                    out_cp[r].wait()
                    out_cp[r] = None
                in_cp[r] = pltpu.async_copy(
                    x_hbm.at[base + b], bufs[r], isems[r])
            if b >= 1:
                pb = b - 1
                pr = pb % NBUF
                in_cp[pr].wait()
                ev = [rows_v[pb, pl.ds(j * L, L)] for j in range(D // L)]
                buf = bufs[pr]

                def add_row(i, _, buf=buf, ev=ev):
                    for j in range(D // L):
                        sl = pl.ds(j * L, L)
                        buf[i, sl] = buf[i, sl] + ev[j]
                    return 0

                lax.fori_loop(0, S, add_row, 0)
                out_cp[pr] = pltpu.async_copy(
                    bufs[pr], out_hbm.at[base + pb], osems[pr])
        for r in range(NBUF):
            if out_cp[r] is not None:
                out_cp[r].wait()

    return body(x, y, emb)


def kernel(x, y, emb):
    y = y.astype(jnp.int32)
    return _sc_full(x, y, emb)
